# baseline (speedup 1.0000x reference)
import jax
import jax.numpy as jnp
from jax import lax
from jax.experimental import pallas as pl

_N = 8192
_NQ = 1024
_K = 16
_QB = 128
_B = 4


_S = 2
_NS = _N // _S


def _fps_kernel(xyzB_ref, ptsB_ref):
    xs = xyzB_ref[0]
    ys = xyzB_ref[1]
    zs = xyzB_ref[2]
    iota_n = (lax.broadcasted_iota(jnp.int32, (_B, _S, _NS), 1) * _NS
              + lax.broadcasted_iota(jnp.int32, (_B, _S, _NS), 2))
    iota_q = lax.broadcasted_iota(jnp.int32, (_B, _NQ), 1)

    def body(i, carry):
        dists, far, cxv, cyv, czv = carry
        nmask = iota_n == far
        fx = jnp.sum(jnp.where(nmask, xs, 0.0), axis=(1, 2), keepdims=True)
        fy = jnp.sum(jnp.where(nmask, ys, 0.0), axis=(1, 2), keepdims=True)
        fz = jnp.sum(jnp.where(nmask, zs, 0.0), axis=(1, 2), keepdims=True)
        qmask = iota_q == i
        cxv = jnp.where(qmask, fx.reshape(_B, 1), cxv)
        cyv = jnp.where(qmask, fy.reshape(_B, 1), cyv)
        czv = jnp.where(qmask, fz.reshape(_B, 1), czv)
        dx = xs - fx
        dy = ys - fy
        dz = zs - fz
        d = dx * dx + dy * dy + dz * dz
        dists = jnp.minimum(dists, d)
        m = jnp.max(dists, axis=(1, 2), keepdims=True)
        far = jnp.min(jnp.where(dists == m, iota_n, jnp.int32(2**30)),
                      axis=(1, 2), keepdims=True)
        return dists, far, cxv, cyv, czv

    init = (jnp.full((_B, _S, _NS), 1e10, dtype=jnp.float32),
            jnp.zeros((_B, 1, 1), dtype=jnp.int32),
            jnp.zeros((_B, _NQ), dtype=jnp.float32),
            jnp.zeros((_B, _NQ), dtype=jnp.float32),
            jnp.zeros((_B, _NQ), dtype=jnp.float32))
    _, _, cxv, cyv, czv = lax.fori_loop(0, _NQ, body, init)
    ptsB_ref[0] = cxv
    ptsB_ref[1] = cyv
    ptsB_ref[2] = czv


def _knn_kernel(xyz_ref, pts_ref, out_ref):
    xs = xyz_ref[0, 0, :][None, :]
    ys = xyz_ref[0, 1, :][None, :]
    zs = xyz_ref[0, 2, :][None, :]
    qx = pts_ref[0, 0, :][:, None]
    qy = pts_ref[0, 1, :][:, None]
    qz = pts_ref[0, 2, :][:, None]
    x_sq = xs * xs + ys * ys + zs * zs
    q_sq = qx * qx + qy * qy + qz * qz
    bxs = xs.astype(jnp.bfloat16).astype(jnp.float32)
    bys = ys.astype(jnp.bfloat16).astype(jnp.float32)
    bzs = zs.astype(jnp.bfloat16).astype(jnp.float32)
    bqx = qx.astype(jnp.bfloat16).astype(jnp.float32)
    bqy = qy.astype(jnp.bfloat16).astype(jnp.float32)
    bqz = qz.astype(jnp.bfloat16).astype(jnp.float32)
    dot = bqx * bxs + bqy * bys + bqz * bzs
    d2 = (q_sq + x_sq) - 2.0 * dot
    iota_n = lax.broadcasted_iota(jnp.int32, (_QB, _N), 1)
    i_prev = None
    for k in range(_K):
        if k > 0:
            d2 = jnp.where(iota_n == i_prev[:, None], jnp.float32(jnp.inf), d2)
        m = jnp.min(d2, axis=1, keepdims=True)
        i_k = jnp.min(jnp.where(d2 == m, iota_n, jnp.int32(2**30)), axis=1)
        out_ref[0, k, :] = i_k
        i_prev = i_k


@jax.jit
def kernel(xyz):
    b = xyz.shape[0]
    xyzB = jnp.reshape(jnp.transpose(xyz, (2, 0, 1)), (3, b, _S, _NS))

    ptsB = pl.pallas_call(
        _fps_kernel,
        grid=(1,),
        in_specs=[pl.BlockSpec((3, b, _S, _NS), lambda i: (0, 0, 0, 0))],
        out_specs=pl.BlockSpec((3, b, _NQ), lambda i: (0, 0, 0)),
        out_shape=jax.ShapeDtypeStruct((3, b, _NQ), jnp.float32),
    )(xyzB)

    xyzT = jnp.transpose(xyz, (0, 2, 1))
    ptsT = jnp.transpose(ptsB, (1, 0, 2))

    knnT = pl.pallas_call(
        _knn_kernel,
        grid=(b, _NQ // _QB),
        in_specs=[pl.BlockSpec((1, 3, _N), lambda i, j: (i, 0, 0)),
                  pl.BlockSpec((1, 3, _QB), lambda i, j: (i, 0, j))],
        out_specs=pl.BlockSpec((1, _K, _QB), lambda i, j: (i, 0, j)),
        out_shape=jax.ShapeDtypeStruct((b, _K, _NQ), jnp.int32),
    )(xyzT, ptsT)

    idx = jnp.transpose(knnT, (0, 2, 1)).astype(jnp.int64)
    pts = jnp.transpose(ptsB, (1, 2, 0))
    return (idx, pts)


# speedup vs baseline: 1.1323x; 1.1323x over previous
import jax
import jax.numpy as jnp
from jax import lax
from jax.experimental import pallas as pl

_N = 8192
_NQ = 1024
_K = 16
_QB = 128
_B = 4


def _fps_kernel(xyzB_ref, ptsB_ref):
    xs = xyzB_ref[0]
    ys = xyzB_ref[1]
    zs = xyzB_ref[2]
    iota_n = lax.broadcasted_iota(jnp.int32, (_B, _N), 1)
    iota_q = lax.broadcasted_iota(jnp.int32, (_B, _NQ), 1)

    def body(i, carry):
        dists, far, cxv, cyv, czv = carry
        nmask = iota_n == far
        fx = jnp.sum(jnp.where(nmask, xs, 0.0), axis=1, keepdims=True)
        fy = jnp.sum(jnp.where(nmask, ys, 0.0), axis=1, keepdims=True)
        fz = jnp.sum(jnp.where(nmask, zs, 0.0), axis=1, keepdims=True)
        qmask = iota_q == i
        cxv = jnp.where(qmask, fx, cxv)
        cyv = jnp.where(qmask, fy, cyv)
        czv = jnp.where(qmask, fz, czv)
        dx = xs - fx
        dy = ys - fy
        dz = zs - fz
        d = dx * dx + dy * dy + dz * dz
        dists = jnp.minimum(dists, d)
        m = jnp.max(dists, axis=1, keepdims=True)
        far = jnp.min(jnp.where(dists == m, iota_n, jnp.int32(2**30)),
                      axis=1, keepdims=True)
        return dists, far, cxv, cyv, czv

    init = (jnp.full((_B, _N), 1e10, dtype=jnp.float32),
            jnp.zeros((_B, 1), dtype=jnp.int32),
            jnp.zeros((_B, _NQ), dtype=jnp.float32),
            jnp.zeros((_B, _NQ), dtype=jnp.float32),
            jnp.zeros((_B, _NQ), dtype=jnp.float32))
    _, _, cxv, cyv, czv = lax.fori_loop(0, _NQ, body, init)
    ptsB_ref[0] = cxv
    ptsB_ref[1] = cyv
    ptsB_ref[2] = czv


def _knn_kernel(xyz_ref, pts_ref, out_ref):
    xs = xyz_ref[0, 0, :][None, :]
    ys = xyz_ref[0, 1, :][None, :]
    zs = xyz_ref[0, 2, :][None, :]
    qx = pts_ref[0, 0, :][:, None]
    qy = pts_ref[0, 1, :][:, None]
    qz = pts_ref[0, 2, :][:, None]
    x_sq = xs * xs + ys * ys + zs * zs
    q_sq = qx * qx + qy * qy + qz * qz
    bxs = xs.astype(jnp.bfloat16).astype(jnp.float32)
    bys = ys.astype(jnp.bfloat16).astype(jnp.float32)
    bzs = zs.astype(jnp.bfloat16).astype(jnp.float32)
    bqx = qx.astype(jnp.bfloat16).astype(jnp.float32)
    bqy = qy.astype(jnp.bfloat16).astype(jnp.float32)
    bqz = qz.astype(jnp.bfloat16).astype(jnp.float32)
    dot = bqx * bxs + bqy * bys + bqz * bzs
    d2 = (q_sq + x_sq) - 2.0 * dot
    iota_n = lax.broadcasted_iota(jnp.int32, (_QB, _N), 1)
    i_prev = None
    for k in range(_K):
        if k > 0:
            d2 = jnp.where(iota_n == i_prev[:, None], jnp.float32(jnp.inf), d2)
        m = jnp.min(d2, axis=1, keepdims=True)
        i_k = jnp.min(jnp.where(d2 == m, iota_n, jnp.int32(2**30)), axis=1)
        out_ref[0, k, :] = i_k
        i_prev = i_k


@jax.jit
def kernel(xyz):
    b = xyz.shape[0]
    xyzB = jnp.transpose(xyz, (2, 0, 1))

    ptsB = pl.pallas_call(
        _fps_kernel,
        grid=(1,),
        in_specs=[pl.BlockSpec((3, b, _N), lambda i: (0, 0, 0))],
        out_specs=pl.BlockSpec((3, b, _NQ), lambda i: (0, 0, 0)),
        out_shape=jax.ShapeDtypeStruct((3, b, _NQ), jnp.float32),
    )(xyzB)

    xyzT = jnp.transpose(xyz, (0, 2, 1))
    ptsT = jnp.transpose(ptsB, (1, 0, 2))

    knnT = pl.pallas_call(
        _knn_kernel,
        grid=(b, _NQ // _QB),
        in_specs=[pl.BlockSpec((1, 3, _N), lambda i, j: (i, 0, 0)),
                  pl.BlockSpec((1, 3, _QB), lambda i, j: (i, 0, j))],
        out_specs=pl.BlockSpec((1, _K, _QB), lambda i, j: (i, 0, j)),
        out_shape=jax.ShapeDtypeStruct((b, _K, _NQ), jnp.int32),
    )(xyzT, ptsT)

    idx = jnp.transpose(knnT, (0, 2, 1)).astype(jnp.int64)
    pts = jnp.transpose(ptsB, (1, 2, 0))
    return (idx, pts)
